# baseline (speedup 1.0000x reference)
import jax
import jax.numpy as jnp
from jax.experimental import pallas as pl
from jax.experimental.pallas import tpu as pltpu

_R = 8
_L = 128
_B = 512
_G = 4
_J = 128
_N = _B * _L


def _pulse_kernel(off_ref, x_ref, o_ref, xt_ref, m_ref, tt_ref, c1_ref):
    x = x_ref[...]
    xt_ref[...] = x.reshape(_R * _B, _L).T.reshape(_L, _R, _B)

    acc = jnp.zeros((_R, _B), jnp.float32)
    for l in range(_L):
        acc = acc + xt_ref[l]
    tot = acc

    for g in range(_G):
        tt_ref[:, g * _R:(g + 1) * _R] = tot[:, g * _J:(g + 1) * _J].T

    acc2 = jnp.zeros((1, _G * _R), jnp.float32)
    for j in range(_J):
        acc2 = acc2 + tt_ref[j:j + 1, :]
        c1_ref[j:j + 1, :] = acc2
    gl = acc2

    zr = jnp.zeros((1, _R), jnp.float32)
    r1 = jnp.concatenate([zr, gl[:, : 3 * _R]], axis=1)
    r2 = jnp.concatenate([jnp.zeros((1, 2 * _R), jnp.float32),
                          gl[:, : 2 * _R]], axis=1)
    r3 = jnp.concatenate([jnp.zeros((1, 3 * _R), jnp.float32),
                          gl[:, : _R]], axis=1)
    grp = jax.lax.broadcasted_iota(jnp.int32, (1, _G * _R), 1) // _R
    zed = jnp.zeros_like(gl)
    excl = jnp.where(grp == 0, zed,
                     jnp.where(grp == 1, r1,
                               jnp.where(grp == 2, r2 + r1, (r3 + r2) + r1)))

    cumb = c1_ref[...] + excl
    cumb2 = jnp.concatenate(
        [cumb[:, g * _R:(g + 1) * _R].T for g in range(_G)], axis=1)
    carry = jnp.concatenate(
        [jnp.zeros((_R, 1), jnp.float32), cumb2[:, : _B - 1]], axis=1)

    off = off_ref[...]
    acc = jnp.zeros((_R, _B), jnp.float32)
    frac0 = None
    prevfrac = None
    for l in range(_L):
        xv = xt_ref[l]
        acc = acc + xv
        xw = off + (acc + carry)
        frac = xw - jnp.floor(xw)
        if l == 0:
            frac0 = frac
            x0 = xv
        else:
            m_ref[l] = (frac - prevfrac < 0).astype(jnp.bfloat16)
        prevfrac = frac
    prev0 = jnp.concatenate(
        [jnp.zeros((_R, 1), jnp.float32), prevfrac[:, : _B - 1]], axis=1)
    m_ref[0] = (frac0 - prev0 < 0).astype(jnp.bfloat16)
    mnat = m_ref[...].reshape(_L, _R * _B).T.reshape(_R, _N)
    o_ref[...] = jnp.where(mnat != 0, jax.lax.rsqrt(x), jnp.float32(0))


def kernel(upsampled_phase, upsampled_phase_offset):
    n_rows = upsampled_phase.shape[0]
    out = pl.pallas_call(
        _pulse_kernel,
        grid=(n_rows // _R,),
        in_specs=[
            pl.BlockSpec((_R, 1), lambda i: (i, 0)),
            pl.BlockSpec((_R, _N), lambda i: (i, 0)),
        ],
        out_specs=pl.BlockSpec((_R, _N), lambda i: (i, 0)),
        out_shape=jax.ShapeDtypeStruct((n_rows, _N), jnp.float32),
        scratch_shapes=[
            pltpu.VMEM((_L, _R, _B), jnp.float32),
            pltpu.VMEM((_L, _R, _B), jnp.bfloat16),
            pltpu.VMEM((_J, _G * _R), jnp.float32),
            pltpu.VMEM((_J, _G * _R), jnp.float32),
        ],
        compiler_params=pltpu.CompilerParams(
            dimension_semantics=("parallel",),
        ),
    )(upsampled_phase_offset, upsampled_phase)
    return out


# speedup vs baseline: 1.0107x; 1.0107x over previous
import jax
import jax.numpy as jnp
from jax.experimental import pallas as pl
from jax.experimental.pallas import tpu as pltpu

_R = 8
_L = 128
_B = 512
_G = 4
_J = 128
_N = _B * _L


def _pulse_kernel(off_ref, x_ref, o_ref, xt_ref, w_ref, tt_ref, c1_ref):
    x = x_ref[...]
    xt_ref[...] = x.reshape(_R * _B, _L).T.reshape(_L, _R, _B)

    acc = jnp.zeros((_R, _B), jnp.float32)
    for l in range(_L):
        acc = acc + xt_ref[l]
    tot = acc

    for g in range(_G):
        tt_ref[:, g * _R:(g + 1) * _R] = tot[:, g * _J:(g + 1) * _J].T

    acc2 = jnp.zeros((1, _G * _R), jnp.float32)
    for j in range(_J):
        acc2 = acc2 + tt_ref[j:j + 1, :]
        c1_ref[j:j + 1, :] = acc2
    gl = acc2

    zr = jnp.zeros((1, _R), jnp.float32)
    r1 = jnp.concatenate([zr, gl[:, : 3 * _R]], axis=1)
    r2 = jnp.concatenate([jnp.zeros((1, 2 * _R), jnp.float32),
                          gl[:, : 2 * _R]], axis=1)
    r3 = jnp.concatenate([jnp.zeros((1, 3 * _R), jnp.float32),
                          gl[:, : _R]], axis=1)
    grp = jax.lax.broadcasted_iota(jnp.int32, (1, _G * _R), 1) // _R
    zed = jnp.zeros_like(gl)
    excl = jnp.where(grp == 0, zed,
                     jnp.where(grp == 1, r1,
                               jnp.where(grp == 2, r2 + r1, (r3 + r2) + r1)))

    cumb = c1_ref[...] + excl
    cumb2 = jnp.concatenate(
        [cumb[:, g * _R:(g + 1) * _R].T for g in range(_G)], axis=1)
    carry = jnp.concatenate(
        [jnp.zeros((_R, 1), jnp.float32), cumb2[:, : _B - 1]], axis=1)

    off = off_ref[...]
    acc = jnp.zeros((_R, _B), jnp.float32)
    frac0 = None
    prevfrac = None
    for l in range(_L):
        xv = xt_ref[l]
        acc = acc + xv
        xw = off + (acc + carry)
        frac = xw - jnp.floor(xw)
        if l == 0:
            frac0 = frac
            x0 = xv
        else:
            mask = frac - prevfrac < 0
            w_ref[l] = jnp.where(mask, jax.lax.rsqrt(xv), jnp.float32(0))
        prevfrac = frac
    prev0 = jnp.concatenate(
        [jnp.zeros((_R, 1), jnp.float32), prevfrac[:, : _B - 1]], axis=1)
    mask0 = frac0 - prev0 < 0
    w_ref[0] = jnp.where(mask0, jax.lax.rsqrt(x0), jnp.float32(0))
    o_ref[...] = w_ref[...].reshape(_L, _R * _B).T.reshape(_R, _N)


def kernel(upsampled_phase, upsampled_phase_offset):
    n_rows = upsampled_phase.shape[0]
    out = pl.pallas_call(
        _pulse_kernel,
        grid=(n_rows // _R,),
        in_specs=[
            pl.BlockSpec((_R, 1), lambda i: (i, 0)),
            pl.BlockSpec((_R, _N), lambda i: (i, 0)),
        ],
        out_specs=pl.BlockSpec((_R, _N), lambda i: (i, 0)),
        out_shape=jax.ShapeDtypeStruct((n_rows, _N), jnp.float32),
        scratch_shapes=[
            pltpu.VMEM((_L, _R, _B), jnp.float32),
            pltpu.VMEM((_L, _R, _B), jnp.float32),
            pltpu.VMEM((_J, _G * _R), jnp.float32),
            pltpu.VMEM((_J, _G * _R), jnp.float32),
        ],
        compiler_params=pltpu.CompilerParams(
            dimension_semantics=("parallel",),
        ),
    )(upsampled_phase_offset, upsampled_phase)
    return out
